# Initial kernel scaffold; baseline (speedup 1.0000x reference)
import jax
import jax.numpy as jnp
from jax.experimental import pallas as pl


def kernel(edge_index, feature, label, W1, b1, W2, b2, W3, b3, Ws, bs, Wm1, bm1, Wm2, bm2, Wm3, bm3, Wn1, bn1, Wn2, bn2, Wn3, bn3):
    raise NotImplementedError("write your pallas kernel here")


# speedup vs baseline: 20.5430x; 20.5430x over previous
import functools
import math

import jax
import jax.numpy as jnp
from jax import lax
from jax.experimental import pallas as pl
from jax.experimental.pallas import tpu as pltpu

NN = 10000
NB = 20
NPER = 500
P = 512
NE = 320000
EPG = NE // NB
D = 128
NCLS = 10
K1 = math.ceil(0.5 * NPER)
K2 = math.ceil(0.5 * K1)
K3 = math.ceil(0.5 * K2)
DUMP = P * P
NEG = -1e30


def _t2row(ide, c):
    return (ide * c).sum(axis=0, keepdims=True)


def _t2col(ide, r):
    return (ide * r).sum(axis=1, keepdims=True)


def _mega_body(a_ref, x_ref, w1, b1, w2, b2, w3, b3, ws, bs,
               wn1, bn1, wn2, bn2, wn3, bn3, vec_out, np_out):
    A = a_ref[0]
    X = x_ref[0]
    f32 = jnp.float32
    rowi = lax.broadcasted_iota(jnp.int32, (P, 1), 0)
    coli = lax.broadcasted_iota(jnp.int32, (1, P), 1)
    ide = (rowi == coli).astype(f32)
    ltm = (coli < rowi)
    valid_c = rowi < NPER
    valid_r = coli < NPER
    neginf = f32(-jnp.inf)
    wsp = ws[...]
    bsv = bs[0, 0]

    def level(x_in, m_cb, m_rb, w, b, k):
        m_c = m_cb.astype(f32)
        m_r = m_rb.astype(f32)
        deg_in = (A * m_r).sum(axis=1, keepdims=True) * m_c
        deg_out_r = (A * m_c).sum(axis=0, keepdims=True) * m_r
        ni = 1.0 / jnp.sqrt(jnp.maximum(deg_in, 1.0))
        no = 1.0 / jnp.sqrt(jnp.maximum(_t2col(ide, deg_out_r), 1.0))
        Z = x_in * (no * m_c)
        agg = jnp.dot(A, Z, preferred_element_type=f32, precision=lax.Precision.HIGHEST) * m_c
        out = jnp.maximum(jnp.dot(agg * ni, w, preferred_element_type=f32)
                          + b, 0.0) * m_c
        sagg = jnp.dot(A, out * no, preferred_element_type=f32,
                       precision=lax.Precision.HIGHEST) * m_c
        score_c = jnp.dot(sagg * ni, wsp,
                          preferred_element_type=f32)[:, 0:1] + bsv
        score_r = _t2row(ide, score_c)
        sm_c = jnp.where(m_cb, score_c, neginf)
        sm_r = jnp.where(m_rb, score_r, neginf)
        beats_i = (sm_r > sm_c) | ((sm_r == sm_c) & ltm)
        beats_j = (sm_c > sm_r) | ((sm_c == sm_r) & (~ltm) & (rowi != coli))
        rank_c = beats_i.astype(jnp.int32).sum(axis=1, keepdims=True)
        rank_r = beats_j.astype(jnp.int32).sum(axis=0, keepdims=True)
        kept_cb = rank_c < k
        kept_rb = rank_r < k
        com_cb = m_cb & (~kept_cb)
        fdis = out * jnp.tanh(score_c)
        return out, score_c, fdis, kept_cb, kept_rb, com_cb, rank_r

    def readout(f, mb, cnt):
        mf = mb.astype(f32)
        avg = (f * mf).sum(axis=0, keepdims=True) * (1.0 / float(cnt))
        mx = jnp.where(mb, f, NEG).max(axis=0, keepdims=True)
        return jnp.concatenate([avg, mx], axis=1)

    out1, score1, fdis1, k1c, k1r, com1, _ = level(
        X, valid_c, valid_r, w1[...], b1[...], K1)
    hg1 = readout(fdis1, k1c, K1)
    hg1c = readout(fdis1, com1, NPER - K1)
    x2 = fdis1 * k1c.astype(f32)
    out2, score2, fdis2, k2c, k2r, com2, rank2r = level(
        x2, k1c, k1r, w2[...], b2[...], K2)
    hg2 = readout(fdis2, k2c, K2)
    hg2c = readout(fdis2, com2, K1 - K2)
    x3 = fdis2 * k2c.astype(f32)
    out3, score3, fdis3, k3c, k3r, com3, _ = level(
        x3, k2c, k2r, w3[...], b3[...], K3)
    hg3 = readout(fdis3, k3c, K3)
    hg3c = readout(fdis3, com3, K2 - K3)

    hg = hg1 + hg2 + hg3
    hgc = hg1c + hg2c + hg3c

    pio = lax.broadcasted_iota(jnp.int32, (D, P), 0)
    onehot = (pio == rank2r).astype(f32)
    g3 = jnp.dot(onehot, out3, preferred_element_type=f32, precision=lax.Precision.HIGHEST)
    h = jnp.maximum(jnp.dot(g3, wn1[...], preferred_element_type=f32)
                    + bn1[...], 0.0)
    h = jnp.maximum(jnp.dot(h, wn2[...], preferred_element_type=f32)
                    + bn2[...], 0.0)
    npred = jnp.dot(h, wn3[...], preferred_element_type=f32) + bn3[...]
    np_out[0] = npred

    z256 = jnp.zeros((1, P - 2 * D), f32)
    z512 = jnp.zeros((4, P), f32)
    rows = jnp.concatenate([
        _t2row(ide, score1),
        jnp.concatenate([hg, z256], axis=1),
        jnp.concatenate([hgc, z256], axis=1),
        jnp.concatenate([hg3c, z256], axis=1),
    ], axis=0)
    vec_out[0] = jnp.concatenate([rows, z512], axis=0)


def _epilogue_body(vec_ref, wm1, bm1, wm2, bm2, wm3, bm3,
                   sc_out, scc_out, ns_out):
    f32 = jnp.float32
    v = vec_ref[...]

    def mlp(h):
        h = jnp.maximum(jnp.dot(h, wm1[...], preferred_element_type=f32)
                        + bm1[...], 0.0)
        h = jnp.maximum(jnp.dot(h, wm2[...], preferred_element_type=f32)
                        + bm2[...], 0.0)
        return jnp.dot(h, wm3[...], preferred_element_type=f32) + bm3[...]

    hg = v[:, 1, :2 * D]
    hgc = v[:, 2, :2 * D]
    sc_out[...] = mlp(hg)
    scc_out[...] = mlp(hgc)

    s = v[:, 0, :]
    mask = lax.broadcasted_iota(jnp.int32, (NB, P), 1) < NPER
    smx = jnp.max(jnp.where(mask, s, f32(-jnp.inf)))
    e = jnp.where(mask, jnp.exp(s - smx), 0.0)
    ns_out[...] = e * (1.0 / jnp.sum(e))


def _full(shape):
    zero = (0,) * len(shape)
    return pl.BlockSpec(shape, lambda g, _z=zero: _z)


@jax.jit
def _run_tc(A, fp, w1, b1, w2, b2, w3, b3, ws, bs,
            wn1, bn1, wn2, bn2, wn3, bn3,
            wm1, bm1, wm2, bm2, wm3, bm3):
    f32 = jnp.float32
    vec, npred = pl.pallas_call(
        _mega_body,
        grid=(NB,),
        in_specs=[
            pl.BlockSpec((1, P, P), lambda g: (g, 0, 0)),
            pl.BlockSpec((1, P, D), lambda g: (g, 0, 0)),
            _full((D, D)), _full((1, D)),
            _full((D, D)), _full((1, D)),
            _full((D, D)), _full((1, D)),
            _full((D, D)), _full((1, 1)),
            _full((D, D)), _full((1, D)),
            _full((D, D)), _full((1, D)),
            _full((D, D)), _full((1, D)),
        ],
        out_specs=[
            pl.BlockSpec((1, 8, P), lambda g: (g, 0, 0)),
            pl.BlockSpec((1, D, D), lambda g: (g, 0, 0)),
        ],
        out_shape=[
            jax.ShapeDtypeStruct((NB, 8, P), f32),
            jax.ShapeDtypeStruct((NB, D, D), f32),
        ],
    )(A, fp, w1, b1, w2, b2, w3, b3, ws, bs,
      wn1, bn1, wn2, bn2, wn3, bn3)

    scores_p, scoresc_p, ns = pl.pallas_call(
        _epilogue_body,
        in_specs=[
            pl.BlockSpec((NB, 8, P), lambda: (0, 0, 0)),
            pl.BlockSpec((2 * D, D), lambda: (0, 0)),
            pl.BlockSpec((1, D), lambda: (0, 0)),
            pl.BlockSpec((D, D), lambda: (0, 0)),
            pl.BlockSpec((1, D), lambda: (0, 0)),
            pl.BlockSpec((D, D), lambda: (0, 0)),
            pl.BlockSpec((1, D), lambda: (0, 0)),
        ],
        out_specs=[
            pl.BlockSpec((NB, D), lambda: (0, 0)),
            pl.BlockSpec((NB, D), lambda: (0, 0)),
            pl.BlockSpec((NB, P), lambda: (0, 0)),
        ],
        out_shape=[
            jax.ShapeDtypeStruct((NB, D), f32),
            jax.ShapeDtypeStruct((NB, D), f32),
            jax.ShapeDtypeStruct((NB, P), f32),
        ],
    )(vec, wm1, bm1, wm2, bm2, wm3, bm3)
    return vec, npred, scores_p, scoresc_p, ns


def _build_adj(flat, g):
    glob = g * (P * P) + flat
    return jnp.zeros((NB * P * P,), jnp.float32).at[glob].add(1.0).reshape(
        NB, P, P)


def kernel(edge_index, feature, label, W1, b1, W2, b2, W3, b3, Ws, bs,
           Wm1, bm1, Wm2, bm2, Wm3, bm3, Wn1, bn1, Wn2, bn2, Wn3, bn3):
    f32 = jnp.float32
    src = edge_index[0].astype(jnp.int32)
    dst = edge_index[1].astype(jnp.int32)
    g = dst // NPER
    flat = (dst - g * NPER) * P + (src - (src // NPER) * NPER)

    A = _build_adj(flat, g)

    fp = jnp.concatenate(
        [feature.reshape(NB, NPER, D), jnp.zeros((NB, P - NPER, D), f32)],
        axis=1)

    def padw(w, r, c):
        return jnp.zeros((r, c), f32).at[:w.shape[0], :w.shape[1]].set(w)

    def padb(v, c):
        return jnp.zeros((1, c), f32).at[0, :v.shape[0]].set(v)

    vec, npred, scores_p, scoresc_p, ns = _run_tc(
        A, fp,
        W1, b1.reshape(1, D), W2, b2.reshape(1, D), W3, b3.reshape(1, D),
        padw(Ws, D, D), bs.reshape(1, 1),
        padw(Wn1, D, D), padb(bn1, D),
        padw(Wn2, D, D), padb(bn2, D),
        padw(Wn3, D, D), padb(bn3, D),
        Wm1, bm1.reshape(1, D),
        padw(Wm2, D, D), padb(bm2, D),
        padw(Wm3, D, D), padb(bm3, D))

    scores = scores_p[:, :NCLS]
    scores_com = scoresc_p[:, :NCLS]
    hg3c = vec[:, 3, :2 * D]
    node_pred = npred[:, :K2, :NCLS].reshape(NB * K2, NCLS)
    node_score1 = ns[:, :NPER].reshape(NN)
    return scores, scores_com, hg3c, node_pred, node_score1
